# Initial kernel scaffold; baseline (speedup 1.0000x reference)
import functools

import jax
import jax.numpy as jnp
from jax import lax
from jax.experimental import pallas as pl
from jax.experimental.pallas import tpu as pltpu
from jax.experimental.pallas import tpu_sc as plsc

NC = 2
NS = 16
CH = 80
KDEPTH = 10


def _sc_accumulate(f0, f1, src2d, cidx2d, zrows, n_nodes, n_chunks):
    two_n = 2 * n_nodes
    rows_per_tile = two_n // NS
    chunks_per_tile = n_chunks // NS
    n_blocks = chunks_per_tile // KDEPTH
    mesh = plsc.VectorSubcoreMesh(core_axis_name="c", subcore_axis_name="s")

    @functools.partial(
        pl.kernel,
        out_type=jax.ShapeDtypeStruct((NC, two_n, 64), jnp.float32),
        mesh=mesh,
        scratch_types=[
            pltpu.VMEM((KDEPTH, CH), jnp.int32),
            pltpu.VMEM((KDEPTH, CH), jnp.int32),
            pltpu.VMEM((KDEPTH, CH, 64), jnp.float32),
            pltpu.VMEM_SHARED((two_n, 64), jnp.float32),
            pltpu.SemaphoreType.DMA,
            pltpu.SemaphoreType.DMA,
        ],
    )
    def k(f0_hbm, f1_hbm, src_hbm, cidx_hbm, zer_hbm, out_hbm,
          idx_s, idx_c, rows, acc, gsem, ssem):
        c = lax.axis_index("c")
        s = lax.axis_index("s")

        pltpu.sync_copy(zer_hbm, acc.at[pl.ds(s * rows_per_tile, rows_per_tile)])
        plsc.subcore_barrier()

        def main(f_hbm):
            def body(i, _):
                blk = s * chunks_per_tile + i * KDEPTH
                pltpu.sync_copy(src_hbm.at[pl.ds(blk, KDEPTH)], idx_s)
                pltpu.sync_copy(cidx_hbm.at[pl.ds(blk, KDEPTH)], idx_c)
                gds = [
                    pltpu.async_copy(f_hbm.at[idx_s.at[j]], rows.at[j], gsem)
                    for j in range(KDEPTH)
                ]
                for d in gds:
                    d.wait()
                sds = [
                    pltpu.async_copy(rows.at[j], acc.at[idx_c.at[j]], ssem,
                                     add=True)
                    for j in range(KDEPTH)
                ]
                for d in sds:
                    d.wait()
                return 0
            lax.fori_loop(0, n_blocks, body, 0)

        @pl.when(c == 0)
        def _():
            main(f0_hbm)

        @pl.when(c == 1)
        def _():
            main(f1_hbm)

        plsc.subcore_barrier()
        pltpu.sync_copy(
            acc.at[pl.ds(s * rows_per_tile, rows_per_tile)],
            out_hbm.at[c, pl.ds(s * rows_per_tile, rows_per_tile)],
        )

    return k(f0, f1, src2d, cidx2d, zrows)


def _tc_tail_body(acc_ref, lin_ref, mw_ref, mb_ref, g_ref, b_ref, out_ref,
                  *, n_nodes, bn_eps):
    n = n_nodes
    h = jnp.zeros((n, 128), dtype=jnp.float32)
    for k in range(2):
        for c in range(2):
            a = acc_ref[c, k * n:(k + 1) * n, :]
            w = lin_ref[k, c * 64:(c + 1) * 64, :]
            h = h + jnp.dot(a, w, preferred_element_type=jnp.float32)
    z = jnp.dot(h, mw_ref[...].T, preferred_element_type=jnp.float32) + mb_ref[...]
    r = jnp.maximum(z, 0.0)
    mean = jnp.mean(r, axis=0, keepdims=True)
    var = jnp.mean((r - mean) * (r - mean), axis=0, keepdims=True)
    out_ref[...] = g_ref[...] * (r - mean) * lax.rsqrt(var + bn_eps) + b_ref[...]


def kernel(feature, sp_embeddings, edge_index, edge_order, linear, mlp_w,
           mlp_b, bn_gamma, bn_beta):
    n_nodes, in_feat = feature.shape
    e = edge_index.shape[1]
    assert in_feat == 128 and e % (NS * CH * KDEPTH) == 0
    assert (2 * n_nodes) % (NS * 8) == 0

    src = edge_index[0]
    cidx = edge_index[1] + edge_order * n_nodes
    n_chunks = e // CH
    src2d = src.reshape(n_chunks, CH)
    cidx2d = cidx.reshape(n_chunks, CH)
    f0 = feature[:, :64]
    f1 = feature[:, 64:]
    zrows = jnp.zeros((2 * n_nodes // NS, 64), dtype=jnp.float32)

    acc = _sc_accumulate(f0, f1, src2d, cidx2d, zrows, n_nodes, n_chunks)

    tail = pl.pallas_call(
        functools.partial(_tc_tail_body, n_nodes=n_nodes, bn_eps=1e-5),
        out_shape=jax.ShapeDtypeStruct((n_nodes, 128), jnp.float32),
    )
    return tail(acc, linear, mlp_w, mlp_b.reshape(1, 128),
                bn_gamma.reshape(1, 128), bn_beta.reshape(1, 128))


# speedup vs baseline: 4.7542x; 4.7542x over previous
import functools

import jax
import jax.numpy as jnp
from jax import lax
from jax.experimental import pallas as pl
from jax.experimental.pallas import tpu as pltpu
from jax.experimental.pallas import tpu_sc as plsc

NC = 2
NS = 16
CH = 128
KDEPTH = 4


def _sc_accumulate(f0, f1, src3d, cidx3d, zrows, two_n_pad, blocks_per_tile):
    rows_per_tile = two_n_pad // NS
    mesh = plsc.VectorSubcoreMesh(core_axis_name="c", subcore_axis_name="s")

    @functools.partial(
        pl.kernel,
        out_type=jax.ShapeDtypeStruct((NC, two_n_pad, 64), jnp.float32),
        mesh=mesh,
        scratch_types=[
            pltpu.VMEM((KDEPTH, CH), jnp.int32),
            pltpu.VMEM((KDEPTH, CH), jnp.int32),
            pltpu.VMEM((KDEPTH, CH, 64), jnp.float32),
            pltpu.VMEM_SHARED((two_n_pad, 64), jnp.float32),
            pltpu.SemaphoreType.DMA,
            pltpu.SemaphoreType.DMA,
        ],
        compiler_params=pltpu.CompilerParams(use_tc_tiling_on_sc=False),
    )
    def k(f0_hbm, f1_hbm, src_hbm, cidx_hbm, zer_hbm, out_hbm,
          idx_s, idx_c, rows, acc, gsem, ssem):
        c = lax.axis_index("c")
        s = lax.axis_index("s")

        pltpu.sync_copy(zer_hbm, acc.at[pl.ds(s * rows_per_tile, rows_per_tile)])
        plsc.subcore_barrier()

        def main(f_hbm):
            def body(i, _):
                blkid = s * blocks_per_tile + i
                pltpu.sync_copy(src_hbm.at[blkid], idx_s)
                pltpu.sync_copy(cidx_hbm.at[blkid], idx_c)
                gds = [
                    pltpu.async_copy(f_hbm.at[idx_s.at[j]], rows.at[j], gsem)
                    for j in range(KDEPTH)
                ]
                for d in gds:
                    d.wait()
                sds = [
                    pltpu.async_copy(rows.at[j], acc.at[idx_c.at[j]], ssem,
                                     add=True)
                    for j in range(KDEPTH)
                ]
                for d in sds:
                    d.wait()
                return 0
            lax.fori_loop(0, blocks_per_tile, body, 0)

        @pl.when(c == 0)
        def _():
            main(f0_hbm)

        @pl.when(c == 1)
        def _():
            main(f1_hbm)

        plsc.subcore_barrier()
        pltpu.sync_copy(
            acc.at[pl.ds(s * rows_per_tile, rows_per_tile)],
            out_hbm.at[c, pl.ds(s * rows_per_tile, rows_per_tile)],
        )

    return k(f0, f1, src3d, cidx3d, zrows)


def _tc_tail_body(acc_ref, lin_ref, mw_ref, mb_ref, g_ref, b_ref, out_ref,
                  *, n_nodes, bn_eps):
    n = n_nodes
    h = jnp.zeros((n, 128), dtype=jnp.float32)
    for k in range(2):
        for c in range(2):
            a = acc_ref[c, k * n:(k + 1) * n, :]
            w = lin_ref[k, c * 64:(c + 1) * 64, :]
            h = h + jnp.dot(a, w, preferred_element_type=jnp.float32)
    z = jnp.dot(h, mw_ref[...].T, preferred_element_type=jnp.float32) + mb_ref[...]
    r = jnp.maximum(z, 0.0)
    mean = jnp.mean(r, axis=0, keepdims=True)
    var = jnp.mean((r - mean) * (r - mean), axis=0, keepdims=True)
    out_ref[...] = g_ref[...] * (r - mean) * lax.rsqrt(var + bn_eps) + b_ref[...]


def kernel(feature, sp_embeddings, edge_index, edge_order, linear, mlp_w,
           mlp_b, bn_gamma, bn_beta):
    n_nodes, in_feat = feature.shape
    e = edge_index.shape[1]
    assert in_feat == 128

    two_n_pad = ((2 * n_nodes + NS * 8) // (NS * 8)) * (NS * 8)

    blk_edges = NS * CH * KDEPTH
    e_pad = ((e + blk_edges - 1) // blk_edges) * blk_edges
    src = edge_index[0]
    cidx = edge_index[1] + edge_order * n_nodes
    if e_pad != e:
        pad = e_pad - e
        src = jnp.concatenate([src, jnp.zeros((pad,), jnp.int32)])
        cidx = jnp.concatenate(
            [cidx, jnp.full((pad,), 2 * n_nodes, jnp.int32)])
    blocks_per_tile = e_pad // blk_edges
    src3d = src.reshape(NS * blocks_per_tile, KDEPTH, CH)
    cidx3d = cidx.reshape(NS * blocks_per_tile, KDEPTH, CH)
    f0 = feature[:, :64]
    f1 = feature[:, 64:]
    zrows = jnp.zeros((two_n_pad // NS, 64), dtype=jnp.float32)

    acc = _sc_accumulate(f0, f1, src3d, cidx3d, zrows, two_n_pad,
                         blocks_per_tile)

    tail = pl.pallas_call(
        functools.partial(_tc_tail_body, n_nodes=n_nodes, bn_eps=1e-5),
        out_shape=jax.ShapeDtypeStruct((n_nodes, 128), jnp.float32),
    )
    return tail(acc, linear, mlp_w, mlp_b.reshape(1, 128),
                bn_gamma.reshape(1, 128), bn_beta.reshape(1, 128))


# speedup vs baseline: 4.9618x; 1.0437x over previous
import functools

import jax
import jax.numpy as jnp
from jax import lax
from jax.experimental import pallas as pl
from jax.experimental.pallas import tpu as pltpu
from jax.experimental.pallas import tpu_sc as plsc

NC = 2
NS = 16
CH = 64
KDEPTH = 4


def _sc_accumulate(f0, f1, src3d, cidx3d, zrows, two_n_pad, blocks_per_tile):
    rows_per_tile = two_n_pad // NS
    nb = blocks_per_tile
    mesh = plsc.VectorSubcoreMesh(core_axis_name="c", subcore_axis_name="s")

    @functools.partial(
        pl.kernel,
        out_type=jax.ShapeDtypeStruct((NC, two_n_pad, 64), jnp.float32),
        mesh=mesh,
        scratch_types=[
            pltpu.VMEM((2, KDEPTH, CH), jnp.int32),
            pltpu.VMEM((2, KDEPTH, CH), jnp.int32),
            pltpu.VMEM((2, KDEPTH, CH, 64), jnp.float32),
            pltpu.VMEM_SHARED((two_n_pad, 64), jnp.float32),
            pltpu.SemaphoreType.DMA,
            pltpu.SemaphoreType.DMA,
            pltpu.SemaphoreType.DMA,
        ],
        compiler_params=pltpu.CompilerParams(use_tc_tiling_on_sc=False),
    )
    def k(f0_hbm, f1_hbm, src_hbm, cidx_hbm, zer_hbm, out_hbm,
          idx_s, idx_c, rows, acc, gsem, ssem, isem):
        c = lax.axis_index("c")
        s = lax.axis_index("s")

        pltpu.sync_copy(zer_hbm, acc.at[pl.ds(s * rows_per_tile, rows_per_tile)])
        plsc.subcore_barrier()

        def main(f_hbm):
            def fire_gathers(m, blkid):
                for j in range(KDEPTH):
                    pltpu.async_copy(f_hbm.at[idx_s.at[m].at[j]],
                                     rows.at[m].at[j], gsem)
                del blkid

            def drain_gathers(m):
                for j in range(KDEPTH):
                    pltpu.make_async_copy(f_hbm.at[idx_s.at[m].at[j]],
                                          rows.at[m].at[j], gsem).wait()

            def fire_scatters(m):
                for j in range(KDEPTH):
                    pltpu.async_copy(rows.at[m].at[j],
                                     acc.at[idx_c.at[m].at[j]], ssem, add=True)

            def drain_scatters(m):
                for j in range(KDEPTH):
                    pltpu.make_async_copy(rows.at[m].at[j],
                                          acc.at[idx_c.at[m].at[j]],
                                          ssem).wait()

            def load_idx(m, blkid):
                a = pltpu.async_copy(src_hbm.at[blkid], idx_s.at[m], isem)
                b = pltpu.async_copy(cidx_hbm.at[blkid], idx_c.at[m], isem)
                return a, b

            i0, i1 = load_idx(0, s * nb)
            i0.wait()
            i1.wait()
            fire_gathers(0, s * nb)

            def phase(i, m):
                drain_gathers(m)

                @pl.when(i >= 1)
                def _():
                    drain_scatters(1 - m)

                @pl.when(i <= nb - 2)
                def _():
                    a, b = load_idx(1 - m, s * nb + i + 1)
                    fire_scatters(m)
                    a.wait()
                    b.wait()
                    fire_gathers(1 - m, s * nb + i + 1)

                @pl.when(i == nb - 1)
                def _():
                    fire_scatters(m)

            def body(i2, _):
                phase(2 * i2, 0)
                phase(2 * i2 + 1, 1)
                return 0

            lax.fori_loop(0, nb // 2, body, 0)
            drain_scatters(1)

        @pl.when(c == 0)
        def _():
            main(f0_hbm)

        @pl.when(c == 1)
        def _():
            main(f1_hbm)

        plsc.subcore_barrier()
        pltpu.sync_copy(
            acc.at[pl.ds(s * rows_per_tile, rows_per_tile)],
            out_hbm.at[c, pl.ds(s * rows_per_tile, rows_per_tile)],
        )

    return k(f0, f1, src3d, cidx3d, zrows)


def _tc_tail_body(acc_ref, lin_ref, mw_ref, mb_ref, g_ref, b_ref, out_ref,
                  *, n_nodes, bn_eps):
    n = n_nodes
    h = jnp.zeros((n, 128), dtype=jnp.float32)
    for k in range(2):
        for c in range(2):
            a = acc_ref[c, k * n:(k + 1) * n, :]
            w = lin_ref[k, c * 64:(c + 1) * 64, :]
            h = h + jnp.dot(a, w, preferred_element_type=jnp.float32)
    z = jnp.dot(h, mw_ref[...].T, preferred_element_type=jnp.float32) + mb_ref[...]
    r = jnp.maximum(z, 0.0)
    mean = jnp.mean(r, axis=0, keepdims=True)
    var = jnp.mean((r - mean) * (r - mean), axis=0, keepdims=True)
    out_ref[...] = g_ref[...] * (r - mean) * lax.rsqrt(var + bn_eps) + b_ref[...]


def kernel(feature, sp_embeddings, edge_index, edge_order, linear, mlp_w,
           mlp_b, bn_gamma, bn_beta):
    n_nodes, in_feat = feature.shape
    e = edge_index.shape[1]
    assert in_feat == 128

    two_n_pad = ((2 * n_nodes + NS * 8) // (NS * 8)) * (NS * 8)

    blk_edges = NS * CH * KDEPTH * 2
    e_pad = ((e + blk_edges - 1) // blk_edges) * blk_edges
    src = edge_index[0]
    cidx = edge_index[1] + edge_order * n_nodes
    if e_pad != e:
        pad = e_pad - e
        src = jnp.concatenate([src, jnp.zeros((pad,), jnp.int32)])
        cidx = jnp.concatenate(
            [cidx, jnp.full((pad,), 2 * n_nodes, jnp.int32)])
    blocks_per_tile = e_pad // (NS * CH * KDEPTH)
    src3d = src.reshape(NS * blocks_per_tile, KDEPTH, CH)
    cidx3d = cidx.reshape(NS * blocks_per_tile, KDEPTH, CH)
    f0 = feature[:, :64]
    f1 = feature[:, 64:]
    zrows = jnp.zeros((two_n_pad // NS, 64), dtype=jnp.float32)

    acc = _sc_accumulate(f0, f1, src3d, cidx3d, zrows, two_n_pad,
                         blocks_per_tile)

    tail = pl.pallas_call(
        functools.partial(_tc_tail_body, n_nodes=n_nodes, bn_eps=1e-5),
        out_shape=jax.ShapeDtypeStruct((n_nodes, 128), jnp.float32),
    )
    return tail(acc, linear, mlp_w, mlp_b.reshape(1, 128),
                bn_gamma.reshape(1, 128), bn_beta.reshape(1, 128))


# speedup vs baseline: 5.4976x; 1.1080x over previous
import functools

import jax
import jax.numpy as jnp
from jax import lax
from jax.experimental import pallas as pl
from jax.experimental.pallas import tpu as pltpu
from jax.experimental.pallas import tpu_sc as plsc

NC = 2
NS = 16
CH = 128
KDEPTH = 2


def _sc_accumulate(f0, f1, idx3d, zrows, two_n_pad, blocks_per_tile):
    rows_per_tile = two_n_pad // NS
    nb = blocks_per_tile
    mesh = plsc.VectorSubcoreMesh(core_axis_name="c", subcore_axis_name="s")

    @functools.partial(
        pl.kernel,
        out_type=jax.ShapeDtypeStruct((NC, two_n_pad, 64), jnp.float32),
        mesh=mesh,
        scratch_types=[
            pltpu.VMEM((2, 2 * KDEPTH, CH), jnp.int32),
            pltpu.VMEM((2, KDEPTH, CH, 64), jnp.float32),
            pltpu.VMEM_SHARED((two_n_pad, 64), jnp.float32),
            pltpu.SemaphoreType.DMA,
            pltpu.SemaphoreType.DMA,
            pltpu.SemaphoreType.DMA,
        ],
        compiler_params=pltpu.CompilerParams(use_tc_tiling_on_sc=False),
    )
    def k(f0_hbm, f1_hbm, idx_hbm, zer_hbm, out_hbm,
          idx, rows, acc, gsem, ssem, isem):
        c = lax.axis_index("c")
        s = lax.axis_index("s")

        pltpu.sync_copy(zer_hbm, acc.at[pl.ds(s * rows_per_tile, rows_per_tile)])
        plsc.subcore_barrier()

        def main(f_hbm):
            def fire_gathers(m):
                for j in range(KDEPTH):
                    pltpu.async_copy(f_hbm.at[idx.at[m].at[j]],
                                     rows.at[m].at[j], gsem)

            def drain_gathers(m):
                for j in range(KDEPTH):
                    pltpu.make_async_copy(f_hbm.at[idx.at[m].at[j]],
                                          rows.at[m].at[j], gsem).wait()

            def fire_scatters(m):
                for j in range(KDEPTH):
                    pltpu.async_copy(rows.at[m].at[j],
                                     acc.at[idx.at[m].at[KDEPTH + j]], ssem,
                                     add=True)

            def drain_scatters(m):
                for j in range(KDEPTH):
                    pltpu.make_async_copy(rows.at[m].at[j],
                                          acc.at[idx.at[m].at[KDEPTH + j]],
                                          ssem).wait()

            def load_idx(m, blkid):
                return pltpu.async_copy(idx_hbm.at[blkid], idx.at[m], isem)

            load_idx(0, s * nb).wait()
            fire_gathers(0)

            def phase(i, m):
                drain_gathers(m)

                @pl.when(i >= 1)
                def _():
                    drain_scatters(1 - m)

                @pl.when(i <= nb - 2)
                def _():
                    a = load_idx(1 - m, s * nb + i + 1)
                    fire_scatters(m)
                    a.wait()
                    fire_gathers(1 - m)

                @pl.when(i == nb - 1)
                def _():
                    fire_scatters(m)

            def body(i2, _):
                phase(2 * i2, 0)
                phase(2 * i2 + 1, 1)
                return 0

            lax.fori_loop(0, nb // 2, body, 0)
            drain_scatters(1)

        @pl.when(c == 0)
        def _():
            main(f0_hbm)

        @pl.when(c == 1)
        def _():
            main(f1_hbm)

        plsc.subcore_barrier()
        pltpu.sync_copy(
            acc.at[pl.ds(s * rows_per_tile, rows_per_tile)],
            out_hbm.at[c, pl.ds(s * rows_per_tile, rows_per_tile)],
        )

    return k(f0, f1, idx3d, zrows)


def _tc_tail_body(acc_ref, lin_ref, mw_ref, mb_ref, g_ref, b_ref, out_ref,
                  *, n_nodes, bn_eps):
    n = n_nodes
    h = jnp.zeros((n, 128), dtype=jnp.float32)
    for k in range(2):
        for c in range(2):
            a = acc_ref[c, k * n:(k + 1) * n, :]
            w = lin_ref[k, c * 64:(c + 1) * 64, :]
            h = h + jnp.dot(a, w, preferred_element_type=jnp.float32)
    z = jnp.dot(h, mw_ref[...].T, preferred_element_type=jnp.float32) + mb_ref[...]
    r = jnp.maximum(z, 0.0)
    mean = jnp.mean(r, axis=0, keepdims=True)
    var = jnp.mean((r - mean) * (r - mean), axis=0, keepdims=True)
    out_ref[...] = g_ref[...] * (r - mean) * lax.rsqrt(var + bn_eps) + b_ref[...]


def kernel(feature, sp_embeddings, edge_index, edge_order, linear, mlp_w,
           mlp_b, bn_gamma, bn_beta):
    n_nodes, in_feat = feature.shape
    e = edge_index.shape[1]
    assert in_feat == 128

    two_n_pad = ((2 * n_nodes + NS * 8) // (NS * 8)) * (NS * 8)

    blk_edges = NS * CH * KDEPTH * 2
    e_pad = ((e + blk_edges - 1) // blk_edges) * blk_edges
    src = edge_index[0]
    cidx = edge_index[1] + edge_order * n_nodes
    if e_pad != e:
        pad = e_pad - e
        src = jnp.concatenate([src, jnp.zeros((pad,), jnp.int32)])
        cidx = jnp.concatenate(
            [cidx, jnp.full((pad,), 2 * n_nodes, jnp.int32)])
    blocks_per_tile = e_pad // (NS * CH * KDEPTH)
    nblk = NS * blocks_per_tile
    idx3d = jnp.concatenate(
        [src.reshape(nblk, KDEPTH, CH), cidx.reshape(nblk, KDEPTH, CH)],
        axis=1)
    f0 = feature[:, :64]
    f1 = feature[:, 64:]
    zrows = jnp.zeros((two_n_pad // NS, 64), dtype=jnp.float32)

    acc = _sc_accumulate(f0, f1, idx3d, zrows, two_n_pad, blocks_per_tile)

    tail = pl.pallas_call(
        functools.partial(_tc_tail_body, n_nodes=n_nodes, bn_eps=1e-5),
        out_shape=jax.ShapeDtypeStruct((n_nodes, 128), jnp.float32),
    )
    return tail(acc, linear, mlp_w, mlp_b.reshape(1, 128),
                bn_gamma.reshape(1, 128), bn_beta.reshape(1, 128))
